# Initial kernel scaffold; baseline (speedup 1.0000x reference)
import jax
import jax.numpy as jnp
from jax.experimental import pallas as pl


def kernel(hidden_states, gate_w, w1, w3, w2):
    raise NotImplementedError("write your pallas kernel here")


# speedup vs baseline: 1.1805x; 1.1805x over previous
import functools

import jax
import jax.numpy as jnp
from jax.experimental import pallas as pl
from jax.experimental.pallas import tpu as pltpu

T = 64
H = 1024
F = 4096
E = 8
TOP_K = 2

BF = 512
NF = F // BF


def _moe_kernel(x_ref, gw_ref, w1_ref, w3_ref, w2_ref, out_ref, comb_ref):
    e = pl.program_id(0)
    f = pl.program_id(1)

    @pl.when((e == 0) & (f == 0))
    def _router():
        x = x_ref[...]
        logits = jax.lax.dot_general(
            x, gw_ref[...], (((1,), (0,)), ((), ())),
            preferred_element_type=jnp.float32,
        )
        m = jnp.max(logits, axis=-1, keepdims=True)
        p = jnp.exp(logits - m)
        rw = p / jnp.sum(p, axis=-1, keepdims=True)
        eidx = jax.lax.broadcasted_iota(jnp.int32, rw.shape, 1)
        v1 = jnp.max(rw, axis=-1, keepdims=True)
        i1 = jnp.min(jnp.where(rw == v1, eidx, E), axis=-1, keepdims=True)
        masked = jnp.where(eidx == i1, -jnp.inf, rw)
        v2 = jnp.max(masked, axis=-1, keepdims=True)
        i2 = jnp.min(jnp.where(masked == v2, eidx, E), axis=-1, keepdims=True)
        keep = (eidx == i1) | (eidx == i2)
        kept = jnp.where(keep, rw, 0.0)
        comb_ref[...] = kept / jnp.sum(kept, axis=-1, keepdims=True)
        out_ref[...] = jnp.zeros_like(out_ref)

    x = x_ref[...]
    w1b = w1_ref[0]
    w3b = w3_ref[0]
    w2b = w2_ref[0]
    g = jax.lax.dot_general(
        x, w1b, (((1,), (1,)), ((), ())), preferred_element_type=jnp.float32)
    u = jax.lax.dot_general(
        x, w3b, (((1,), (1,)), ((), ())), preferred_element_type=jnp.float32)
    eidx = jax.lax.broadcasted_iota(jnp.int32, (T, E), 1)
    c = jnp.sum(jnp.where(eidx == e, comb_ref[...], 0.0), axis=-1,
                keepdims=True)
    act = (g * jax.nn.sigmoid(g)) * u * c
    out_ref[...] += jax.lax.dot_general(
        act, w2b, (((1,), (1,)), ((), ())), preferred_element_type=jnp.float32)


@functools.partial(jax.jit, static_argnames=("interpret",))
def kernel(hidden_states, gate_w, w1, w3, w2, interpret=False):
    return pl.pallas_call(
        _moe_kernel,
        grid=(E, NF),
        in_specs=[
            pl.BlockSpec((T, H), lambda e, f: (0, 0)),
            pl.BlockSpec((H, E), lambda e, f: (0, 0)),
            pl.BlockSpec((1, BF, H), lambda e, f: (e, f, 0)),
            pl.BlockSpec((1, BF, H), lambda e, f: (e, f, 0)),
            pl.BlockSpec((1, H, BF), lambda e, f: (e, 0, f)),
        ],
        out_specs=pl.BlockSpec((T, H), lambda e, f: (0, 0)),
        out_shape=jax.ShapeDtypeStruct((T, H), jnp.float32),
        scratch_shapes=[pltpu.VMEM((T, E), jnp.float32)],
        compiler_params=pltpu.CompilerParams(
            dimension_semantics=("arbitrary", "arbitrary"),
        ),
        interpret=interpret,
    )(hidden_states, gate_w, w1, w3, w2)


# speedup vs baseline: 1.3289x; 1.1257x over previous
import functools

import jax
import jax.numpy as jnp
from jax.experimental import pallas as pl
from jax.experimental.pallas import tpu as pltpu

T = 64
H = 1024
F = 4096
E = 8
TOP_K = 2

BF = 1024
NF = F // BF


def _moe_kernel(x_ref, gw_ref, w1_ref, w3_ref, w2_ref, out_ref, comb_ref):
    e = pl.program_id(0)
    f = pl.program_id(1)

    @pl.when((e == 0) & (f == 0))
    def _router():
        x = x_ref[...]
        logits = jax.lax.dot_general(
            x, gw_ref[...], (((1,), (0,)), ((), ())),
            preferred_element_type=jnp.float32,
        )
        m = jnp.max(logits, axis=-1, keepdims=True)
        p = jnp.exp(logits - m)
        rw = p / jnp.sum(p, axis=-1, keepdims=True)
        eidx = jax.lax.broadcasted_iota(jnp.int32, rw.shape, 1)
        v1 = jnp.max(rw, axis=-1, keepdims=True)
        i1 = jnp.min(jnp.where(rw == v1, eidx, E), axis=-1, keepdims=True)
        masked = jnp.where(eidx == i1, -jnp.inf, rw)
        v2 = jnp.max(masked, axis=-1, keepdims=True)
        i2 = jnp.min(jnp.where(masked == v2, eidx, E), axis=-1, keepdims=True)
        keep = (eidx == i1) | (eidx == i2)
        kept = jnp.where(keep, rw, 0.0)
        comb_ref[...] = kept / jnp.sum(kept, axis=-1, keepdims=True)
        out_ref[...] = jnp.zeros_like(out_ref)

    x = x_ref[...]
    w1b = w1_ref[0]
    w3b = w3_ref[0]
    w2b = w2_ref[0]
    g = jax.lax.dot_general(
        x, w1b, (((1,), (1,)), ((), ())), preferred_element_type=jnp.float32)
    u = jax.lax.dot_general(
        x, w3b, (((1,), (1,)), ((), ())), preferred_element_type=jnp.float32)
    eidx = jax.lax.broadcasted_iota(jnp.int32, (T, E), 1)
    c = jnp.sum(jnp.where(eidx == e, comb_ref[...], 0.0), axis=-1,
                keepdims=True)
    act = (g * jax.nn.sigmoid(g)) * u * c
    out_ref[...] += jax.lax.dot_general(
        act, w2b, (((1,), (1,)), ((), ())), preferred_element_type=jnp.float32)


@functools.partial(jax.jit, static_argnames=("interpret",))
def kernel(hidden_states, gate_w, w1, w3, w2, interpret=False):
    return pl.pallas_call(
        _moe_kernel,
        grid=(E, NF),
        in_specs=[
            pl.BlockSpec((T, H), lambda e, f: (0, 0)),
            pl.BlockSpec((H, E), lambda e, f: (0, 0)),
            pl.BlockSpec((1, BF, H), lambda e, f: (e, f, 0)),
            pl.BlockSpec((1, BF, H), lambda e, f: (e, f, 0)),
            pl.BlockSpec((1, H, BF), lambda e, f: (e, 0, f)),
        ],
        out_specs=pl.BlockSpec((T, H), lambda e, f: (0, 0)),
        out_shape=jax.ShapeDtypeStruct((T, H), jnp.float32),
        scratch_shapes=[pltpu.VMEM((T, E), jnp.float32)],
        compiler_params=pltpu.CompilerParams(
            dimension_semantics=("arbitrary", "arbitrary"),
        ),
        interpret=interpret,
    )(hidden_states, gate_w, w1, w3, w2)
